# baseline (speedup 1.0000x reference)
import functools

import jax
import jax.numpy as jnp
from jax.experimental import pallas as pl
from jax.experimental.pallas import tpu as pltpu

B, D, H, N = 16384, 64, 64, 64
GATES = 4 * H
BB = 64


def _cell_kernel(x_ref, slots_ref, cum_ref, delta_ref, filled_ref,
                 wqt_ref, wk_ref, wvt_ref, bv_ref, a12_ref, a3_ref,
                 whht_ref, bias_ref,
                 h_out_ref, slots_out_ref, cum_out_ref, delta_out_ref,
                 filled_out_ref):
    x = x_ref[...]
    slots = slots_ref[...]
    cum = cum_ref[...]
    delta = delta_ref[...]
    filled = filled_ref[...]

    f32 = jnp.float32
    iota3 = jax.lax.broadcasted_iota(jnp.int32, (BB, N, 1), 1)

    q = jnp.dot(x, wqt_ref[...], preferred_element_type=f32)
    qk = jnp.dot(q, wk_ref[...], preferred_element_type=f32)
    sims = jnp.sum(slots * qk[:, None, :], axis=2, keepdims=True)

    empty = filled == 0.0
    idx_empty = jnp.min(jnp.where(empty, iota3, N), axis=1, keepdims=True)
    sims_max = jnp.max(sims, axis=1, keepdims=True)
    idx_cont = jnp.min(jnp.where(sims == sims_max, iota3, N),
                       axis=1, keepdims=True)
    idx = jnp.where(idx_empty < N, idx_empty, idx_cont)
    onehot = iota3 == idx

    v = jnp.dot(x, wvt_ref[...], preferred_element_type=f32) + bv_ref[...]
    delta_new = jnp.where(onehot, 0.0, delta + 1.0)
    cum_new = jnp.where(onehot, x[:, None, :], cum + x[:, None, :])
    slots_new = jnp.where(onehot, v[:, None, :], slots)
    filled_new = jnp.where(onehot, 1.0, filled)

    slots_out_ref[...] = slots_new
    cum_out_ref[...] = cum_new
    delta_out_ref[...] = delta_new
    filled_out_ref[...] = filled_new

    x2 = jnp.concatenate(
        [slots_new.reshape(BB * N, D), cum_new.reshape(BB * N, D)], axis=1)
    gin = jnp.dot(x2, a12_ref[...], preferred_element_type=f32)
    gin3 = (gin.reshape(BB, N, GATES)
            + delta_new * a3_ref[...][None]
            + bias_ref[...][None])

    whht = whht_ref[...]
    h = jnp.zeros((BB, H), dtype=f32)
    c = jnp.zeros((BB, H), dtype=f32)
    for t in range(N):
        g = gin3[:, t, :] + jnp.dot(h, whht, preferred_element_type=f32)
        i_g = jax.nn.sigmoid(g[:, 0:H])
        f_g = jax.nn.sigmoid(g[:, H:2 * H])
        g_g = jnp.tanh(g[:, 2 * H:3 * H])
        o_g = jax.nn.sigmoid(g[:, 3 * H:4 * H])
        c = f_g * c + i_g * g_g
        h = o_g * jnp.tanh(c)
    h_out_ref[...] = h


@functools.partial(jax.jit, static_argnames=("interpret",))
def _run(x_t, slots, cum_feats, delta3, filled3,
         wqt, wk, wvt, bv, a12, a3, whht, bias, interpret=False):
    grid = (B // BB,)
    row = lambda i: (i, 0)
    row3 = lambda i: (i, 0, 0)
    rep = lambda i: (0, 0)
    in_specs = [
        pl.BlockSpec((BB, D), row),
        pl.BlockSpec((BB, N, D), row3),
        pl.BlockSpec((BB, N, D), row3),
        pl.BlockSpec((BB, N, 1), row3),
        pl.BlockSpec((BB, N, 1), row3),
        pl.BlockSpec((D, H), rep),
        pl.BlockSpec((H, D), rep),
        pl.BlockSpec((D, D), rep),
        pl.BlockSpec((1, D), rep),
        pl.BlockSpec((2 * D, GATES), rep),
        pl.BlockSpec((1, GATES), rep),
        pl.BlockSpec((H, GATES), rep),
        pl.BlockSpec((1, GATES), rep),
    ]
    out_specs = [
        pl.BlockSpec((BB, H), row),
        pl.BlockSpec((BB, N, D), row3),
        pl.BlockSpec((BB, N, D), row3),
        pl.BlockSpec((BB, N, 1), row3),
        pl.BlockSpec((BB, N, 1), row3),
    ]
    out_shapes = [
        jax.ShapeDtypeStruct((B, H), jnp.float32),
        jax.ShapeDtypeStruct((B, N, D), jnp.float32),
        jax.ShapeDtypeStruct((B, N, D), jnp.float32),
        jax.ShapeDtypeStruct((B, N, 1), jnp.float32),
        jax.ShapeDtypeStruct((B, N, 1), jnp.float32),
    ]
    return pl.pallas_call(
        _cell_kernel,
        grid=grid,
        in_specs=in_specs,
        out_specs=out_specs,
        out_shape=out_shapes,
        compiler_params=pltpu.CompilerParams(
            dimension_semantics=("arbitrary",)),
        interpret=interpret,
    )(x_t, slots, cum_feats, delta3, filled3,
      wqt, wk, wvt, bv, a12, a3, whht, bias)


def kernel(x_t, h_mem_prev, slots, cum_feats, delta_t, filled,
           W_q, W_k, W_v, b_v, W_ih, W_hh, b_ih, b_hh):
    del h_mem_prev
    delta3 = delta_t.reshape(B, N, 1)
    filled3 = filled.astype(jnp.float32).reshape(B, N, 1)
    wqt = W_q.T
    wvt = W_v.T
    bv = b_v.reshape(1, D)
    a12 = W_ih[:, :2 * D].T
    a3 = W_ih[:, 2 * D].reshape(1, GATES)
    whht = W_hh.T
    bias = (b_ih + b_hh).reshape(1, GATES)
    h_mem, slots_o, cum_o, delta_o, filled_o = _run(
        x_t, slots, cum_feats, delta3, filled3,
        wqt, W_k, wvt, bv, a12, a3, whht, bias)
    return (h_mem, slots_o, cum_o, delta_o.reshape(B, N),
            filled_o.reshape(B, N) > 0.5)


# speedup vs baseline: 1.5369x; 1.5369x over previous
import functools

import jax
import jax.numpy as jnp
from jax.experimental import pallas as pl
from jax.experimental.pallas import tpu as pltpu

B, D, H, N = 16384, 64, 64, 64
GATES = 4 * H
BB = 128


def _cell_kernel(x_ref, slots_ref, cum_ref, deltac_ref, delta_ref,
                 filled_ref, wqt_ref, wk_ref, wvt_ref, bv_ref, a12_ref,
                 a3_ref, whht_ref, bias_ref,
                 h_out_ref, slots_out_ref, cum_out_ref, delta_out_ref,
                 filled_out_ref):
    x = x_ref[...]
    slots = slots_ref[...]
    cum = cum_ref[...]
    deltac = deltac_ref[...]
    delta = delta_ref[...]
    filled = filled_ref[...]

    f32 = jnp.float32
    iota3 = jax.lax.broadcasted_iota(jnp.int32, (BB, N, 1), 1)
    iota13 = jax.lax.broadcasted_iota(jnp.int32, (BB, 1, N), 2)

    q = jnp.dot(x, wqt_ref[...], preferred_element_type=f32)
    qk = jnp.dot(q, wk_ref[...], preferred_element_type=f32)
    sims = jnp.sum(slots * qk[:, None, :], axis=2, keepdims=True)

    empty = filled == 0.0
    idx_empty = jnp.min(jnp.where(empty, iota13, N), axis=2, keepdims=True)
    sims_max = jnp.max(sims, axis=1, keepdims=True)
    idx_cont = jnp.min(jnp.where(sims == sims_max, iota3, N),
                       axis=1, keepdims=True)
    idx = jnp.where(idx_empty < N, idx_empty, idx_cont)
    onehot = iota3 == idx
    onehot13 = iota13 == idx

    v = jnp.dot(x, wvt_ref[...], preferred_element_type=f32) + bv_ref[...]
    deltac_new = jnp.where(onehot, 0.0, deltac + 1.0)
    cum_new = jnp.where(onehot, x[:, None, :], cum + x[:, None, :])
    slots_new = jnp.where(onehot, v[:, None, :], slots)

    slots_out_ref[...] = slots_new
    cum_out_ref[...] = cum_new
    delta_out_ref[...] = jnp.where(onehot13, 0.0, delta + 1.0)
    filled_out_ref[...] = jnp.where(onehot13, 1.0, filled)

    lane = jax.lax.broadcasted_iota(jnp.int32, (1, GATES), 1)
    gsel = jnp.logical_and(lane >= 2 * H, lane < 3 * H)
    gscale = jnp.where(gsel, 2.0, 1.0)
    a12 = a12_ref[...] * gscale
    a3 = a3_ref[...] * gscale
    whht = whht_ref[...] * gscale
    bias = bias_ref[...] * gscale

    x2 = jnp.concatenate(
        [slots_new.reshape(BB * N, D), cum_new.reshape(BB * N, D)], axis=1)
    gin = jnp.dot(x2, a12, preferred_element_type=f32)
    gin3 = (gin.reshape(BB, N, GATES)
            + deltac_new * a3[None]
            + bias[None])

    h = jnp.zeros((BB, H), dtype=f32)
    c = jnp.zeros((BB, H), dtype=f32)
    for t in range(N):
        g = gin3[:, t, :] + jnp.dot(h, whht, preferred_element_type=f32)
        sig = jax.nn.sigmoid(g)
        i_g = sig[:, 0:H]
        f_g = sig[:, H:2 * H]
        g_g = 2.0 * sig[:, 2 * H:3 * H] - 1.0
        o_g = sig[:, 3 * H:4 * H]
        c = f_g * c + i_g * g_g
        h = o_g * jnp.tanh(c)
    h_out_ref[...] = h


@functools.partial(jax.jit, static_argnames=("interpret",))
def _run(x_t, slots, cum_feats, deltac3, delta13, filled13,
         wqt, wk, wvt, bv, a12, a3, whht, bias, interpret=False):
    grid = (B // BB,)
    row = lambda i: (i, 0)
    row3 = lambda i: (i, 0, 0)
    rep = lambda i: (0, 0)
    in_specs = [
        pl.BlockSpec((BB, D), row),
        pl.BlockSpec((BB, N, D), row3),
        pl.BlockSpec((BB, N, D), row3),
        pl.BlockSpec((BB, N, 1), row3),
        pl.BlockSpec((BB, 1, N), row3),
        pl.BlockSpec((BB, 1, N), row3),
        pl.BlockSpec((D, H), rep),
        pl.BlockSpec((H, D), rep),
        pl.BlockSpec((D, D), rep),
        pl.BlockSpec((1, D), rep),
        pl.BlockSpec((2 * D, GATES), rep),
        pl.BlockSpec((1, GATES), rep),
        pl.BlockSpec((H, GATES), rep),
        pl.BlockSpec((1, GATES), rep),
    ]
    out_specs = [
        pl.BlockSpec((BB, H), row),
        pl.BlockSpec((BB, N, D), row3),
        pl.BlockSpec((BB, N, D), row3),
        pl.BlockSpec((BB, 1, N), row3),
        pl.BlockSpec((BB, 1, N), row3),
    ]
    out_shapes = [
        jax.ShapeDtypeStruct((B, H), jnp.float32),
        jax.ShapeDtypeStruct((B, N, D), jnp.float32),
        jax.ShapeDtypeStruct((B, N, D), jnp.float32),
        jax.ShapeDtypeStruct((B, 1, N), jnp.float32),
        jax.ShapeDtypeStruct((B, 1, N), jnp.float32),
    ]
    return pl.pallas_call(
        _cell_kernel,
        grid=grid,
        in_specs=in_specs,
        out_specs=out_specs,
        out_shape=out_shapes,
        compiler_params=pltpu.CompilerParams(
            dimension_semantics=("arbitrary",)),
        interpret=interpret,
    )(x_t, slots, cum_feats, deltac3, delta13, filled13,
      wqt, wk, wvt, bv, a12, a3, whht, bias)


def kernel(x_t, h_mem_prev, slots, cum_feats, delta_t, filled,
           W_q, W_k, W_v, b_v, W_ih, W_hh, b_ih, b_hh):
    del h_mem_prev
    deltac3 = delta_t.reshape(B, N, 1)
    delta13 = delta_t.reshape(B, 1, N)
    filled13 = filled.astype(jnp.float32).reshape(B, 1, N)
    wqt = W_q.T
    wvt = W_v.T
    bv = b_v.reshape(1, D)
    a12 = W_ih[:, :2 * D].T
    a3 = W_ih[:, 2 * D].reshape(1, GATES)
    whht = W_hh.T
    bias = (b_ih + b_hh).reshape(1, GATES)
    h_mem, slots_o, cum_o, delta_o, filled_o = _run(
        x_t, slots, cum_feats, deltac3, delta13, filled13,
        wqt, W_k, wvt, bv, a12, a3, whht, bias)
    return (h_mem, slots_o, cum_o, delta_o.reshape(B, N),
            filled_o.reshape(B, N) > 0.5)


# speedup vs baseline: 2.6802x; 1.7439x over previous
import functools

import jax
import jax.numpy as jnp
from jax.experimental import pallas as pl
from jax.experimental.pallas import tpu as pltpu

B, D, H, N = 16384, 64, 64, 64
GATES = 4 * H
BB = 128
BH = BB // 2


def _cell_kernel(x_ref, slots_ref, cum_ref, deltac_ref, delta_ref,
                 filled_ref, wqt_ref, wk_ref, wvt_ref, bv_ref, a12p_ref,
                 a3p_ref, w2p_ref, biasp_ref,
                 h_out_ref, slots_out_ref, cum_out_ref, delta_out_ref,
                 filled_out_ref):
    x = x_ref[...]
    slots = slots_ref[...]
    cum = cum_ref[...]
    deltac = deltac_ref[...]
    delta = delta_ref[...]
    filled = filled_ref[...]

    f32 = jnp.float32
    iota3 = jax.lax.broadcasted_iota(jnp.int32, (BB, N, 1), 1)
    iota13 = jax.lax.broadcasted_iota(jnp.int32, (BB, 1, N), 2)

    q = jnp.dot(x, wqt_ref[...], preferred_element_type=f32)
    qk = jnp.dot(q, wk_ref[...], preferred_element_type=f32)
    sims = jnp.sum(slots * qk[:, None, :], axis=2, keepdims=True)

    empty = filled == 0.0
    idx_empty = jnp.min(jnp.where(empty, iota13, N), axis=2, keepdims=True)
    sims_max = jnp.max(sims, axis=1, keepdims=True)
    idx_cont = jnp.min(jnp.where(sims == sims_max, iota3, N),
                       axis=1, keepdims=True)
    idx = jnp.where(idx_empty < N, idx_empty, idx_cont)
    onehot = iota3 == idx
    onehot13 = iota13 == idx

    v = jnp.dot(x, wvt_ref[...], preferred_element_type=f32) + bv_ref[...]
    deltac_new = jnp.where(onehot, 0.0, deltac + 1.0)
    cum_new = jnp.where(onehot, x[:, None, :], cum + x[:, None, :])
    slots_new = jnp.where(onehot, v[:, None, :], slots)

    slots_out_ref[...] = slots_new
    cum_out_ref[...] = cum_new
    delta_out_ref[...] = jnp.where(onehot13, 0.0, delta + 1.0)
    filled_out_ref[...] = jnp.where(onehot13, 1.0, filled)

    lane = jax.lax.broadcasted_iota(jnp.int32, (1, 2 * GATES), 1)
    gsel = jnp.logical_and(lane >= 4 * H, lane < 6 * H)
    gscale = jnp.where(gsel, 2.0, 1.0)
    a12p = a12p_ref[...] * gscale
    a3p = a3p_ref[...] * gscale
    w2p = w2p_ref[...] * gscale
    biasp = biasp_ref[...] * gscale
    lane_a = (lane % (2 * H)) < H
    a3p_a = jnp.where(lane_a, a3p, 0.0)
    a3p_b = jnp.where(lane_a, 0.0, a3p)

    x2 = jnp.concatenate(
        [slots_new.reshape(BB * N, D), cum_new.reshape(BB * N, D)], axis=1)
    x2p = jnp.concatenate([x2[:BH * N], x2[BH * N:]], axis=1)
    ginp = jnp.dot(x2p, a12p, preferred_element_type=f32)
    ginp3 = (ginp.reshape(BH, N, 2 * GATES)
             + deltac_new[:BH] * a3p_a[None]
             + deltac_new[BH:] * a3p_b[None]
             + biasp[None])
    gin_t = jnp.transpose(ginp3, (1, 0, 2))

    hp = jnp.zeros((BH, 2 * H), dtype=f32)
    cp = jnp.zeros((BH, 2 * H), dtype=f32)
    for t in range(N):
        g = gin_t[t] + jnp.dot(hp, w2p, preferred_element_type=f32)
        sig = jax.nn.sigmoid(g)
        i_p = sig[:, 0:2 * H]
        f_p = sig[:, 2 * H:4 * H]
        g_p = 2.0 * sig[:, 4 * H:6 * H] - 1.0
        o_p = sig[:, 6 * H:8 * H]
        cp = f_p * cp + i_p * g_p
        hp = o_p * jnp.tanh(cp)
    h_out_ref[:BH, :] = hp[:, 0:H]
    h_out_ref[BH:, :] = hp[:, H:2 * H]


@functools.partial(jax.jit, static_argnames=("interpret",))
def _run(x_t, slots, cum_feats, deltac3, delta13, filled13,
         wqt, wk, wvt, bv, a12p, a3p, w2p, biasp, interpret=False):
    grid = (B // BB,)
    row = lambda i: (i, 0)
    row3 = lambda i: (i, 0, 0)
    rep = lambda i: (0, 0)
    in_specs = [
        pl.BlockSpec((BB, D), row),
        pl.BlockSpec((BB, N, D), row3),
        pl.BlockSpec((BB, N, D), row3),
        pl.BlockSpec((BB, N, 1), row3),
        pl.BlockSpec((BB, 1, N), row3),
        pl.BlockSpec((BB, 1, N), row3),
        pl.BlockSpec((D, H), rep),
        pl.BlockSpec((H, D), rep),
        pl.BlockSpec((D, D), rep),
        pl.BlockSpec((1, D), rep),
        pl.BlockSpec((4 * D, 2 * GATES), rep),
        pl.BlockSpec((1, 2 * GATES), rep),
        pl.BlockSpec((2 * H, 2 * GATES), rep),
        pl.BlockSpec((1, 2 * GATES), rep),
    ]
    out_specs = [
        pl.BlockSpec((BB, H), row),
        pl.BlockSpec((BB, N, D), row3),
        pl.BlockSpec((BB, N, D), row3),
        pl.BlockSpec((BB, 1, N), row3),
        pl.BlockSpec((BB, 1, N), row3),
    ]
    out_shapes = [
        jax.ShapeDtypeStruct((B, H), jnp.float32),
        jax.ShapeDtypeStruct((B, N, D), jnp.float32),
        jax.ShapeDtypeStruct((B, N, D), jnp.float32),
        jax.ShapeDtypeStruct((B, 1, N), jnp.float32),
        jax.ShapeDtypeStruct((B, 1, N), jnp.float32),
    ]
    return pl.pallas_call(
        _cell_kernel,
        grid=grid,
        in_specs=in_specs,
        out_specs=out_specs,
        out_shape=out_shapes,
        compiler_params=pltpu.CompilerParams(
            dimension_semantics=("arbitrary",)),
        interpret=interpret,
    )(x_t, slots, cum_feats, deltac3, delta13, filled13,
      wqt, wk, wvt, bv, a12p, a3p, w2p, biasp)


def _pack_cols(w):
    r = w.shape[0]
    wg = w.reshape(r, 4, H)
    z = jnp.zeros_like(wg)
    return jnp.concatenate([wg, z], axis=2).reshape(r, 2 * GATES)


def _pack_cols_b(w):
    r = w.shape[0]
    wg = w.reshape(r, 4, H)
    z = jnp.zeros_like(wg)
    return jnp.concatenate([z, wg], axis=2).reshape(r, 2 * GATES)


def kernel(x_t, h_mem_prev, slots, cum_feats, delta_t, filled,
           W_q, W_k, W_v, b_v, W_ih, W_hh, b_ih, b_hh):
    del h_mem_prev
    deltac3 = delta_t.reshape(B, N, 1)
    delta13 = delta_t.reshape(B, 1, N)
    filled13 = filled.astype(jnp.float32).reshape(B, 1, N)
    wqt = W_q.T
    wvt = W_v.T
    bv = b_v.reshape(1, D)
    a1 = W_ih[:, :D].T
    a2 = W_ih[:, D:2 * D].T
    a3 = W_ih[:, 2 * D].reshape(1, GATES)
    whht = W_hh.T
    bias = (b_ih + b_hh).reshape(1, GATES)
    a12p = jnp.concatenate([
        _pack_cols(a1), _pack_cols(a2),
        _pack_cols_b(a1), _pack_cols_b(a2)], axis=0)
    w2p = jnp.concatenate([
        _pack_cols(whht), _pack_cols_b(whht)], axis=0)
    a3p = _pack_cols(a3) + _pack_cols_b(a3)
    biasp = _pack_cols(bias) + _pack_cols_b(bias)
    h_mem, slots_o, cum_o, delta_o, filled_o = _run(
        x_t, slots, cum_feats, deltac3, delta13, filled13,
        wqt, W_k, wvt, bv, a12p, a3p, w2p, biasp)
    return (h_mem, slots_o, cum_o, delta_o.reshape(B, N),
            filled_o.reshape(B, N) > 0.5)
